# Initial kernel scaffold; baseline (speedup 1.0000x reference)
import functools

import jax
import jax.numpy as jnp
from jax import lax
from jax.experimental import pallas as pl
from jax.experimental.pallas import tpu as pltpu
from jax.experimental.pallas import tpu_sc as plsc

N_NODES = 10000
N_PAD = 10240
N_EDGES = 320000
D = 128
N_CLASSES = 10

NC, NS = 2, 16
NW = NC * NS
EPT = N_EDGES // NW
CHUNK = 80
NCHUNK = EPT // CHUNK
RPT = N_PAD // NS

_mesh = plsc.VectorSubcoreMesh(core_axis_name="c", subcore_axis_name="s")


@functools.partial(
    pl.kernel,
    mesh=_mesh,
    out_type=jax.ShapeDtypeStruct((NW, N_PAD), jnp.float32),
    scratch_types=[
        pltpu.VMEM((EPT,), jnp.int32),
        pltpu.VMEM((N_PAD,), jnp.float32),
    ],
)
def _deg_kernel(dst_hbm, out_hbm, dst_v, hist):
    c = lax.axis_index("c")
    s = lax.axis_index("s")
    wid = c * NS + s
    pltpu.sync_copy(dst_hbm.at[pl.ds(wid * EPT, EPT)], dst_v)

    def zero(i, carry):
        hist[pl.ds(i * 16, 16)] = jnp.zeros((16,), jnp.float32)
        return carry

    lax.fori_loop(0, N_PAD // 16, zero, 0)

    ones = jnp.ones((16,), jnp.float32)

    def body(i, carry):
        dvec = dst_v[pl.ds(i * 16, 16)]
        plsc.addupdate_scatter(hist, [dvec], ones)
        return carry

    lax.fori_loop(0, EPT // 16, body, 0)
    pltpu.sync_copy(hist, out_hbm.at[wid])


@functools.partial(
    pl.kernel,
    mesh=_mesh,
    out_type=jax.ShapeDtypeStruct((NC * N_PAD, D), jnp.float32),
    scratch_types=[
        pltpu.VMEM((CHUNK,), jnp.int32),
        pltpu.VMEM((CHUNK,), jnp.int32),
        pltpu.VMEM((CHUNK, D), jnp.float32),
        pltpu.VMEM_SHARED((N_PAD, D), jnp.float32),
        pltpu.SemaphoreType.DMA,
    ],
)
def _edge_kernel(hn_hbm, src_hbm, dst_hbm, zeros_hbm, out_hbm,
                 idx_s, idx_d, rows, acc, sem):
    c = lax.axis_index("c")
    s = lax.axis_index("s")
    wid = c * NS + s
    pltpu.sync_copy(zeros_hbm, acc.at[pl.ds(s * RPT, RPT)])
    plsc.subcore_barrier()

    def body(i, carry):
        base = pl.multiple_of(wid * EPT + i * CHUNK, 8)
        pltpu.sync_copy(src_hbm.at[pl.ds(base, CHUNK)], idx_s)
        pltpu.sync_copy(dst_hbm.at[pl.ds(base, CHUNK)], idx_d)
        pltpu.async_copy(hn_hbm.at[idx_s], rows, sem).wait()
        pltpu.sync_copy(rows, acc.at[idx_d], add=True)
        return carry

    lax.fori_loop(0, NCHUNK, body, 0)
    plsc.subcore_barrier()
    pltpu.sync_copy(acc.at[pl.ds(s * RPT, RPT)],
                    out_hbm.at[pl.ds(c * N_PAD + s * RPT, RPT)])


def _mm1_body(x_ref, degp_ref, w_ref, o_ref):
    deg = jnp.sum(degp_ref[...], axis=0) + 1.0
    nrm = lax.rsqrt(deg)
    h = jnp.dot(x_ref[...], w_ref[...], preferred_element_type=jnp.float32)
    o_ref[...] = h * nrm[:, None]


def _mm1(x_pad, deg_parts, W1):
    blk = 1024
    grid = N_PAD // blk
    return pl.pallas_call(
        _mm1_body,
        grid=(grid,),
        in_specs=[
            pl.BlockSpec((blk, D), lambda i: (i, 0)),
            pl.BlockSpec((NW, blk), lambda i: (0, i)),
            pl.BlockSpec((D, D), lambda i: (0, 0)),
        ],
        out_specs=pl.BlockSpec((blk, D), lambda i: (i, 0)),
        out_shape=jax.ShapeDtypeStruct((N_PAD, D), jnp.float32),
    )(x_pad, deg_parts, W1)


def _mm2_body(parts_ref, hn_ref, degp_ref, b_ref, w_ref, o_ref):
    deg = jnp.sum(degp_ref[...], axis=0) + 1.0
    nrm = lax.rsqrt(deg)
    agg = parts_ref[0] + parts_ref[1] + hn_ref[...]
    g = jnp.maximum(agg * nrm[:, None] + b_ref[...], 0.0)
    h = jnp.dot(g, w_ref[...], preferred_element_type=jnp.float32)
    o_ref[...] = h * nrm[:, None]


def _mm2(parts, hn, deg_parts, b, W):
    blk = 1024
    grid = N_PAD // blk
    return pl.pallas_call(
        _mm2_body,
        grid=(grid,),
        in_specs=[
            pl.BlockSpec((2, blk, D), lambda i: (0, i, 0)),
            pl.BlockSpec((blk, D), lambda i: (i, 0)),
            pl.BlockSpec((NW, blk), lambda i: (0, i)),
            pl.BlockSpec((1, D), lambda i: (0, 0)),
            pl.BlockSpec((D, D), lambda i: (0, 0)),
        ],
        out_specs=pl.BlockSpec((blk, D), lambda i: (i, 0)),
        out_shape=jax.ShapeDtypeStruct((N_PAD, D), jnp.float32),
    )(parts, hn, deg_parts, b, W)


def _fin_body(parts_ref, hn_ref, degp_ref, b_ref, wc_ref, bc_ref, o_ref):
    i = pl.program_id(0)
    blk = parts_ref.shape[1]
    deg = jnp.sum(degp_ref[...], axis=0) + 1.0
    nrm = lax.rsqrt(deg)
    agg = parts_ref[0] + parts_ref[1] + hn_ref[...]
    g = jnp.maximum(agg * nrm[:, None] + b_ref[...], 0.0)
    rows = lax.broadcasted_iota(jnp.int32, (blk, D), 0) + i * blk
    g = jnp.where(rows < N_NODES, g, 0.0)
    colsum = jnp.sum(g, axis=0, keepdims=True) * (1.0 / N_NODES)

    @pl.when(i == 0)
    def _():
        o_ref[...] = bc_ref[...]

    o_ref[...] += jnp.dot(colsum, wc_ref[...],
                          preferred_element_type=jnp.float32)


def _fin(parts, hn, deg_parts, b, Wc, bc):
    blk = 1024
    grid = N_PAD // blk
    return pl.pallas_call(
        _fin_body,
        grid=(grid,),
        in_specs=[
            pl.BlockSpec((2, blk, D), lambda i: (0, i, 0)),
            pl.BlockSpec((blk, D), lambda i: (i, 0)),
            pl.BlockSpec((NW, blk), lambda i: (0, i)),
            pl.BlockSpec((1, D), lambda i: (0, 0)),
            pl.BlockSpec((D, N_CLASSES), lambda i: (0, 0)),
            pl.BlockSpec((1, N_CLASSES), lambda i: (0, 0)),
        ],
        out_specs=pl.BlockSpec((1, N_CLASSES), lambda i: (0, 0)),
        out_shape=jax.ShapeDtypeStruct((1, N_CLASSES), jnp.float32),
    )(parts, hn, deg_parts, b, Wc, bc)


@jax.jit
def kernel(x, edge_index, W1, b1, W2, b2, Wc, bc):
    src = edge_index[0].astype(jnp.int32)
    dst = edge_index[1].astype(jnp.int32)
    x_pad = jnp.pad(x, ((0, N_PAD - N_NODES), (0, 0)))
    zeros = jnp.zeros((RPT, D), jnp.float32)
    b1r = b1.reshape(1, D)
    b2r = b2.reshape(1, D)
    bcr = bc.reshape(1, N_CLASSES)

    deg_parts = _deg_kernel(dst)
    hn1 = _mm1(x_pad, deg_parts, W1)
    parts1 = _edge_kernel(hn1, src, dst, zeros).reshape(NC, N_PAD, D)
    hn2 = _mm2(parts1, hn1, deg_parts, b1r, W2)
    parts2 = _edge_kernel(hn2, src, dst, zeros).reshape(NC, N_PAD, D)
    return _fin(parts2, hn2, deg_parts, b2r, Wc, bcr)


# speedup vs baseline: 9.7758x; 9.7758x over previous
import functools

import jax
import jax.numpy as jnp
from jax import lax
from jax.experimental import pallas as pl
from jax.experimental.pallas import tpu as pltpu
from jax.experimental.pallas import tpu_sc as plsc

N_NODES = 10000
N_PAD = 10240
N_EDGES = 320000
D = 128
N_CLASSES = 10

NC, NS = 2, 16
NW = NC * NS
EPT = N_EDGES // NW
CHUNK = 80
NCHUNK = EPT // CHUNK
RPT = N_PAD // NS

_mesh = plsc.VectorSubcoreMesh(core_axis_name="c", subcore_axis_name="s")


@functools.partial(
    pl.kernel,
    mesh=_mesh,
    out_type=jax.ShapeDtypeStruct((NC * N_PAD, 16), jnp.float32),
    scratch_types=[
        pltpu.VMEM((CHUNK,), jnp.int32),
        pltpu.VMEM((CHUNK, 16), jnp.float32),
        pltpu.VMEM_SHARED((N_PAD, 16), jnp.float32),
    ],
)
def _deg_kernel(dst_hbm, zeros_hbm, out_hbm, idx_d, ones_buf, acc):
    c = lax.axis_index("c")
    s = lax.axis_index("s")
    wid = c * NS + s
    e0 = jnp.where(lax.iota(jnp.int32, 16) == 0, 1.0, 0.0)

    def fill(i, carry):
        ones_buf[i, :] = e0
        return carry

    lax.fori_loop(0, CHUNK, fill, 0)
    pltpu.sync_copy(zeros_hbm, acc.at[pl.ds(s * RPT, RPT)])
    plsc.subcore_barrier()

    def body(i, carry):
        base = pl.multiple_of(wid * EPT + i * CHUNK, 8)
        pltpu.sync_copy(dst_hbm.at[pl.ds(base, CHUNK)], idx_d)
        pltpu.sync_copy(ones_buf, acc.at[idx_d], add=True)
        return carry

    lax.fori_loop(0, NCHUNK, body, 0)
    plsc.subcore_barrier()
    pltpu.sync_copy(acc.at[pl.ds(s * RPT, RPT)],
                    out_hbm.at[pl.ds(c * N_PAD + s * RPT, RPT)])


@functools.partial(
    pl.kernel,
    mesh=_mesh,
    out_type=jax.ShapeDtypeStruct((NC * N_PAD, D), jnp.float32),
    scratch_types=[
        pltpu.VMEM((CHUNK,), jnp.int32),
        pltpu.VMEM((CHUNK,), jnp.int32),
        pltpu.VMEM((CHUNK, D), jnp.float32),
        pltpu.VMEM_SHARED((N_PAD, D), jnp.float32),
        pltpu.SemaphoreType.DMA,
    ],
)
def _edge_kernel(hn_hbm, src_hbm, dst_hbm, zeros_hbm, out_hbm,
                 idx_s, idx_d, rows, acc, sem):
    c = lax.axis_index("c")
    s = lax.axis_index("s")
    wid = c * NS + s
    pltpu.sync_copy(zeros_hbm, acc.at[pl.ds(s * RPT, RPT)])
    plsc.subcore_barrier()

    def body(i, carry):
        base = pl.multiple_of(wid * EPT + i * CHUNK, 8)
        pltpu.sync_copy(src_hbm.at[pl.ds(base, CHUNK)], idx_s)
        pltpu.sync_copy(dst_hbm.at[pl.ds(base, CHUNK)], idx_d)
        pltpu.async_copy(hn_hbm.at[idx_s], rows, sem).wait()
        pltpu.sync_copy(rows, acc.at[idx_d], add=True)
        return carry

    lax.fori_loop(0, NCHUNK, body, 0)
    plsc.subcore_barrier()
    pltpu.sync_copy(acc.at[pl.ds(s * RPT, RPT)],
                    out_hbm.at[pl.ds(c * N_PAD + s * RPT, RPT)])


def _mm1_body(x_ref, degp_ref, w_ref, o_ref):
    dp = degp_ref[...]
    deg = jnp.sum(dp[0] + dp[1], axis=1, keepdims=True) + 1.0
    nrm = lax.rsqrt(deg)
    h = jnp.dot(x_ref[...], w_ref[...], preferred_element_type=jnp.float32)
    o_ref[...] = h * nrm


def _mm1(x_pad, deg_parts, W1):
    blk = 1024
    grid = N_PAD // blk
    return pl.pallas_call(
        _mm1_body,
        grid=(grid,),
        in_specs=[
            pl.BlockSpec((blk, D), lambda i: (i, 0)),
            pl.BlockSpec((2, blk, 16), lambda i: (0, i, 0)),
            pl.BlockSpec((D, D), lambda i: (0, 0)),
        ],
        out_specs=pl.BlockSpec((blk, D), lambda i: (i, 0)),
        out_shape=jax.ShapeDtypeStruct((N_PAD, D), jnp.float32),
    )(x_pad, deg_parts, W1)


def _mm2_body(parts_ref, hn_ref, degp_ref, b_ref, w_ref, o_ref):
    dp = degp_ref[...]
    deg = jnp.sum(dp[0] + dp[1], axis=1, keepdims=True) + 1.0
    nrm = lax.rsqrt(deg)
    agg = parts_ref[0] + parts_ref[1] + hn_ref[...]
    g = jnp.maximum(agg * nrm + b_ref[...], 0.0)
    h = jnp.dot(g, w_ref[...], preferred_element_type=jnp.float32)
    o_ref[...] = h * nrm


def _mm2(parts, hn, deg_parts, b, W):
    blk = 1024
    grid = N_PAD // blk
    return pl.pallas_call(
        _mm2_body,
        grid=(grid,),
        in_specs=[
            pl.BlockSpec((2, blk, D), lambda i: (0, i, 0)),
            pl.BlockSpec((blk, D), lambda i: (i, 0)),
            pl.BlockSpec((2, blk, 16), lambda i: (0, i, 0)),
            pl.BlockSpec((1, D), lambda i: (0, 0)),
            pl.BlockSpec((D, D), lambda i: (0, 0)),
        ],
        out_specs=pl.BlockSpec((blk, D), lambda i: (i, 0)),
        out_shape=jax.ShapeDtypeStruct((N_PAD, D), jnp.float32),
    )(parts, hn, deg_parts, b, W)


def _fin_body(parts_ref, hn_ref, degp_ref, b_ref, wc_ref, bc_ref, o_ref):
    i = pl.program_id(0)
    blk = parts_ref.shape[1]
    dp = degp_ref[...]
    deg = jnp.sum(dp[0] + dp[1], axis=1, keepdims=True) + 1.0
    nrm = lax.rsqrt(deg)
    agg = parts_ref[0] + parts_ref[1] + hn_ref[...]
    g = jnp.maximum(agg * nrm + b_ref[...], 0.0)
    rows = lax.broadcasted_iota(jnp.int32, (blk, D), 0) + i * blk
    g = jnp.where(rows < N_NODES, g, 0.0)
    colsum = jnp.sum(g, axis=0, keepdims=True) * (1.0 / N_NODES)

    @pl.when(i == 0)
    def _():
        o_ref[...] = bc_ref[...]

    o_ref[...] += jnp.dot(colsum, wc_ref[...],
                          preferred_element_type=jnp.float32)


def _fin(parts, hn, deg_parts, b, Wc, bc):
    blk = 1024
    grid = N_PAD // blk
    return pl.pallas_call(
        _fin_body,
        grid=(grid,),
        in_specs=[
            pl.BlockSpec((2, blk, D), lambda i: (0, i, 0)),
            pl.BlockSpec((blk, D), lambda i: (i, 0)),
            pl.BlockSpec((2, blk, 16), lambda i: (0, i, 0)),
            pl.BlockSpec((1, D), lambda i: (0, 0)),
            pl.BlockSpec((D, N_CLASSES), lambda i: (0, 0)),
            pl.BlockSpec((1, N_CLASSES), lambda i: (0, 0)),
        ],
        out_specs=pl.BlockSpec((1, N_CLASSES), lambda i: (0, 0)),
        out_shape=jax.ShapeDtypeStruct((1, N_CLASSES), jnp.float32),
    )(parts, hn, deg_parts, b, Wc, bc)


@jax.jit
def kernel(x, edge_index, W1, b1, W2, b2, Wc, bc):
    src = edge_index[0].astype(jnp.int32)
    dst = edge_index[1].astype(jnp.int32)
    x_pad = jnp.pad(x, ((0, N_PAD - N_NODES), (0, 0)))
    zeros = jnp.zeros((RPT, D), jnp.float32)
    zeros16 = jnp.zeros((RPT, 16), jnp.float32)
    b1r = b1.reshape(1, D)
    b2r = b2.reshape(1, D)
    bcr = bc.reshape(1, N_CLASSES)

    deg_parts = _deg_kernel(dst, zeros16).reshape(NC, N_PAD, 16)
    hn1 = _mm1(x_pad, deg_parts, W1)
    parts1 = _edge_kernel(hn1, src, dst, zeros).reshape(NC, N_PAD, D)
    hn2 = _mm2(parts1, hn1, deg_parts, b1r, W2)
    parts2 = _edge_kernel(hn2, src, dst, zeros).reshape(NC, N_PAD, D)
    return _fin(parts2, hn2, deg_parts, b2r, Wc, bcr)


# speedup vs baseline: 18.3986x; 1.8821x over previous
import functools

import jax
import jax.numpy as jnp
from jax import lax
from jax.experimental import pallas as pl
from jax.experimental.pallas import tpu as pltpu
from jax.experimental.pallas import tpu_sc as plsc

N_NODES = 10000
N_PAD = 10240
N_EDGES = 320000
D = 128
N_CLASSES = 10

NC, NS = 2, 16
NW = NC * NS
EPT = N_EDGES // NW
CHUNK = 80
NCHUNK = EPT // CHUNK
RPT = N_PAD // NS

_mesh = plsc.VectorSubcoreMesh(core_axis_name="c", subcore_axis_name="s")


@functools.partial(
    pl.kernel,
    mesh=_mesh,
    out_type=jax.ShapeDtypeStruct((NC * N_PAD, 16), jnp.float32),
    scratch_types=[
        pltpu.VMEM((EPT,), jnp.int32),
        pltpu.VMEM((CHUNK, 16), jnp.float32),
        pltpu.VMEM_SHARED((N_PAD, 16), jnp.float32),
        pltpu.SemaphoreType.DMA,
        pltpu.SemaphoreType.DMA,
    ],
)
def _deg_kernel(dst_hbm, zeros_hbm, out_hbm, dsti, ones_buf, acc, s0, s1):
    ssem = [s0, s1]
    c = lax.axis_index("c")
    s = lax.axis_index("s")
    wid = c * NS + s
    e0 = jnp.where(lax.iota(jnp.int32, 16) == 0, 1.0, 0.0)

    def fill(i, carry):
        ones_buf[i, :] = e0
        return carry

    lax.fori_loop(0, CHUNK, fill, 0)

    def isc(g, b):
        pltpu.async_copy(
            ones_buf, acc.at[dsti.at[pl.ds(g * CHUNK, CHUNK)]],
            ssem[b], add=True)

    def ws(b):
        pltpu.make_async_copy(
            ones_buf, acc.at[dsti.at[pl.ds(0, CHUNK)]], ssem[b]).wait()

    base = pl.multiple_of(wid * EPT, 8)
    pltpu.sync_copy(dst_hbm.at[pl.ds(base, EPT)], dsti)
    pltpu.sync_copy(zeros_hbm, acc.at[pl.ds(s * RPT, RPT)])
    plsc.subcore_barrier()

    isc(0, 0)

    def pair(k, carry):
        t = 2 * k + 1
        isc(t, 1)
        ws(0)
        isc(t + 1, 0)
        ws(1)
        return carry

    lax.fori_loop(0, 62, pair, 0)
    ws(0)
    plsc.subcore_barrier()
    pltpu.sync_copy(acc.at[pl.ds(s * RPT, RPT)],
                    out_hbm.at[pl.ds(c * N_PAD + s * RPT, RPT)])


@functools.partial(
    pl.kernel,
    mesh=_mesh,
    out_type=jax.ShapeDtypeStruct((NC * N_PAD, D), jnp.float32),
    scratch_types=[
        pltpu.VMEM((EPT,), jnp.int32),
        pltpu.VMEM((EPT,), jnp.int32),
        pltpu.VMEM((CHUNK, D), jnp.float32),
        pltpu.VMEM((CHUNK, D), jnp.float32),
        pltpu.VMEM_SHARED((N_PAD, D), jnp.float32),
        pltpu.SemaphoreType.DMA,
        pltpu.SemaphoreType.DMA,
        pltpu.SemaphoreType.DMA,
        pltpu.SemaphoreType.DMA,
    ],
)
def _edge_kernel(hn_hbm, src_hbm, dst_hbm, zeros_hbm, out_hbm,
                 srci, dsti, r0, r1, acc, g0, g1, s0, s1):
    rows = [r0, r1]
    gsem = [g0, g1]
    ssem = [s0, s1]
    c = lax.axis_index("c")
    s = lax.axis_index("s")
    wid = c * NS + s

    def ig(g, b):
        pltpu.async_copy(
            hn_hbm.at[srci.at[pl.ds(g * CHUNK, CHUNK)]], rows[b], gsem[b])

    def wg(b):
        pltpu.make_async_copy(
            hn_hbm.at[srci.at[pl.ds(0, CHUNK)]], rows[b], gsem[b]).wait()

    def isc(g, b):
        pltpu.async_copy(
            rows[b], acc.at[dsti.at[pl.ds(g * CHUNK, CHUNK)]],
            ssem[b], add=True)

    def ws(b):
        pltpu.make_async_copy(
            rows[b], acc.at[dsti.at[pl.ds(0, CHUNK)]], ssem[b]).wait()

    base = pl.multiple_of(wid * EPT, 8)
    pltpu.sync_copy(src_hbm.at[pl.ds(base, EPT)], srci)
    pltpu.sync_copy(dst_hbm.at[pl.ds(base, EPT)], dsti)
    pltpu.sync_copy(zeros_hbm, acc.at[pl.ds(s * RPT, RPT)])
    plsc.subcore_barrier()

    ig(0, 0)
    wg(0)
    isc(0, 0)
    ig(1, 1)

    def pair(k, carry):
        t = 2 * k + 1
        wg(1); isc(t, 1); ws(0); ig(t + 1, 0)
        wg(0); isc(t + 1, 0); ws(1); ig(t + 2, 1)
        return carry

    lax.fori_loop(0, 61, pair, 0)
    wg(1); isc(123, 1); ws(0); ig(124, 0)
    wg(0); isc(124, 0); ws(1)
    ws(0)
    plsc.subcore_barrier()
    pltpu.sync_copy(acc.at[pl.ds(s * RPT, RPT)],
                    out_hbm.at[pl.ds(c * N_PAD + s * RPT, RPT)])


def _mm1_body(x_ref, degp_ref, w_ref, o_ref):
    dp = degp_ref[...]
    deg = jnp.sum(dp[0] + dp[1], axis=1, keepdims=True) + 1.0
    nrm = lax.rsqrt(deg)
    h = jnp.dot(x_ref[...], w_ref[...], preferred_element_type=jnp.float32)
    o_ref[...] = h * nrm


def _mm1(x_pad, deg_parts, W1):
    blk = 1024
    grid = N_PAD // blk
    return pl.pallas_call(
        _mm1_body,
        grid=(grid,),
        in_specs=[
            pl.BlockSpec((blk, D), lambda i: (i, 0)),
            pl.BlockSpec((2, blk, 16), lambda i: (0, i, 0)),
            pl.BlockSpec((D, D), lambda i: (0, 0)),
        ],
        out_specs=pl.BlockSpec((blk, D), lambda i: (i, 0)),
        out_shape=jax.ShapeDtypeStruct((N_PAD, D), jnp.float32),
    )(x_pad, deg_parts, W1)


def _mm2_body(parts_ref, hn_ref, degp_ref, b_ref, w_ref, o_ref):
    dp = degp_ref[...]
    deg = jnp.sum(dp[0] + dp[1], axis=1, keepdims=True) + 1.0
    nrm = lax.rsqrt(deg)
    agg = parts_ref[0] + parts_ref[1] + hn_ref[...]
    g = jnp.maximum(agg * nrm + b_ref[...], 0.0)
    h = jnp.dot(g, w_ref[...], preferred_element_type=jnp.float32)
    o_ref[...] = h * nrm


def _mm2(parts, hn, deg_parts, b, W):
    blk = 1024
    grid = N_PAD // blk
    return pl.pallas_call(
        _mm2_body,
        grid=(grid,),
        in_specs=[
            pl.BlockSpec((2, blk, D), lambda i: (0, i, 0)),
            pl.BlockSpec((blk, D), lambda i: (i, 0)),
            pl.BlockSpec((2, blk, 16), lambda i: (0, i, 0)),
            pl.BlockSpec((1, D), lambda i: (0, 0)),
            pl.BlockSpec((D, D), lambda i: (0, 0)),
        ],
        out_specs=pl.BlockSpec((blk, D), lambda i: (i, 0)),
        out_shape=jax.ShapeDtypeStruct((N_PAD, D), jnp.float32),
    )(parts, hn, deg_parts, b, W)


def _fin_body(parts_ref, hn_ref, degp_ref, b_ref, wc_ref, bc_ref, o_ref):
    i = pl.program_id(0)
    blk = parts_ref.shape[1]
    dp = degp_ref[...]
    deg = jnp.sum(dp[0] + dp[1], axis=1, keepdims=True) + 1.0
    nrm = lax.rsqrt(deg)
    agg = parts_ref[0] + parts_ref[1] + hn_ref[...]
    g = jnp.maximum(agg * nrm + b_ref[...], 0.0)
    rows = lax.broadcasted_iota(jnp.int32, (blk, D), 0) + i * blk
    g = jnp.where(rows < N_NODES, g, 0.0)
    colsum = jnp.sum(g, axis=0, keepdims=True) * (1.0 / N_NODES)

    @pl.when(i == 0)
    def _():
        o_ref[...] = bc_ref[...]

    o_ref[...] += jnp.dot(colsum, wc_ref[...],
                          preferred_element_type=jnp.float32)


def _fin(parts, hn, deg_parts, b, Wc, bc):
    blk = 1024
    grid = N_PAD // blk
    return pl.pallas_call(
        _fin_body,
        grid=(grid,),
        in_specs=[
            pl.BlockSpec((2, blk, D), lambda i: (0, i, 0)),
            pl.BlockSpec((blk, D), lambda i: (i, 0)),
            pl.BlockSpec((2, blk, 16), lambda i: (0, i, 0)),
            pl.BlockSpec((1, D), lambda i: (0, 0)),
            pl.BlockSpec((D, N_CLASSES), lambda i: (0, 0)),
            pl.BlockSpec((1, N_CLASSES), lambda i: (0, 0)),
        ],
        out_specs=pl.BlockSpec((1, N_CLASSES), lambda i: (0, 0)),
        out_shape=jax.ShapeDtypeStruct((1, N_CLASSES), jnp.float32),
    )(parts, hn, deg_parts, b, Wc, bc)


@jax.jit
def kernel(x, edge_index, W1, b1, W2, b2, Wc, bc):
    src = edge_index[0].astype(jnp.int32)
    dst = edge_index[1].astype(jnp.int32)
    x_pad = jnp.pad(x, ((0, N_PAD - N_NODES), (0, 0)))
    zeros = jnp.zeros((RPT, D), jnp.float32)
    zeros16 = jnp.zeros((RPT, 16), jnp.float32)
    b1r = b1.reshape(1, D)
    b2r = b2.reshape(1, D)
    bcr = bc.reshape(1, N_CLASSES)

    deg_parts = _deg_kernel(dst, zeros16).reshape(NC, N_PAD, 16)
    hn1 = _mm1(x_pad, deg_parts, W1)
    parts1 = _edge_kernel(hn1, src, dst, zeros).reshape(NC, N_PAD, D)
    hn2 = _mm2(parts1, hn1, deg_parts, b1r, W2)
    parts2 = _edge_kernel(hn2, src, dst, zeros).reshape(NC, N_PAD, D)
    return _fin(parts2, hn2, deg_parts, b2r, Wc, bcr)
